# baseline (speedup 1.0000x reference)
import functools

import jax
import jax.numpy as jnp
from jax import lax
from jax.experimental import pallas as pl

_N = 128
_NB = 8732
_NC = 21
_M = _N * _NB
_B = 4096
_GRID_A = (_M + _B - 1) // _B

_I32_MIN = jnp.iinfo(jnp.int32).min


def _stage_a_body(conf_ref, lab_ref, lp_ref, lt_ref, labrep_ref,
                  cl_ref, accloc_ref, accnll_ref, accnp_ref):
    g = pl.program_id(0)
    zero = jnp.zeros((1, 1), jnp.float32)

    @pl.when(g == 0)
    def _init():
        accloc_ref[...] = zero
        accnll_ref[...] = zero
        accnp_ref[...] = zero

    x = conf_ref[...]
    lab = lab_ref[...]

    idx = lax.broadcasted_iota(jnp.int32, (1, _B), 1) + g * _B
    valid = idx < _M
    pos = (lab > 0) & valid

    m = jnp.max(x, axis=0, keepdims=True)
    e = jnp.exp(x - m)
    s = jnp.sum(e, axis=0, keepdims=True)
    lse = jnp.log(s) + m
    ci = lax.broadcasted_iota(jnp.int32, (_NC, _B), 0)
    pick = jnp.sum(jnp.where(ci == lab, x, 0.0), axis=0, keepdims=True)
    nll = lse - pick

    cl_ref[...] = jnp.where(pos, -1.0, jnp.where(valid, nll, 0.0))

    accnll_ref[...] += jnp.sum(jnp.where(pos, nll, 0.0)).reshape(1, 1)
    accnp_ref[...] += jnp.sum(jnp.where(pos, 1.0, 0.0)).reshape(1, 1)

    idx4 = lax.broadcasted_iota(jnp.int32, (1, 4 * _B), 1) + g * (4 * _B)
    posd = (labrep_ref[...] > 0) & (idx4 < 4 * _M)
    d = lp_ref[...] - lt_ref[...]
    ad = jnp.abs(d)
    sl1 = jnp.where(ad < 1.0, 0.5 * d * d, ad - 0.5)
    accloc_ref[...] += jnp.sum(jnp.where(posd, sl1, 0.0)).reshape(1, 1)


def _stage_b_body(cl_ref, out_ref, *, rows):
    pid = pl.program_id(0)
    x = cl_ref[...]
    i = lax.bitcast_convert_type(x, jnp.int32)
    kb = jnp.where(i >= 0, i, i ^ 0x7FFFFFFF)

    p = jnp.sum((x == -1.0).astype(jnp.int32), axis=1, keepdims=True)
    j = jnp.minimum(jnp.minimum(3 * p, _NB - 1), _NB - p)

    def bit_step(it, prefix):
        t = prefix + (jnp.int32(1) << (31 - it))
        cnt = jnp.sum((kb >= t).astype(jnp.int32), axis=1, keepdims=True)
        return jnp.where(cnt >= j, t, prefix)

    prefix = lax.fori_loop(
        0, 32, bit_step, jnp.full((rows, 1), _I32_MIN, jnp.int32))

    gt = kb > prefix
    c_gt = jnp.sum(gt.astype(jnp.int32), axis=1, keepdims=True)
    sum_gt = jnp.sum(jnp.where(gt, x, 0.0), axis=1, keepdims=True)
    tbits = jnp.where(prefix >= 0, prefix, prefix ^ 0x7FFFFFFF)
    tval = lax.bitcast_convert_type(tbits, jnp.float32)
    row = jnp.where(j > 0, sum_gt + (j - c_gt).astype(jnp.float32) * tval, 0.0)

    @pl.when(pid == 0)
    def _init():
        out_ref[...] = jnp.zeros((1, 1), jnp.float32)

    out_ref[...] += jnp.sum(row).reshape(1, 1)


def kernel(loc_preds, loc_targets, conf_preds, label_targets):
    labels = label_targets.astype(jnp.int32).reshape(1, _M)
    conf_t = conf_preds.reshape(_M, _NC).T
    lpf = loc_preds.reshape(1, 4 * _M)
    ltf = loc_targets.reshape(1, 4 * _M)
    labrep = jnp.repeat(labels.reshape(_M, 1), 4, axis=1).reshape(1, 4 * _M)

    cl, loc_loss, nll_pos, num_pos = pl.pallas_call(
        _stage_a_body,
        grid=(_GRID_A,),
        in_specs=[
            pl.BlockSpec((_NC, _B), lambda g: (0, g)),
            pl.BlockSpec((1, _B), lambda g: (0, g)),
            pl.BlockSpec((1, 4 * _B), lambda g: (0, g)),
            pl.BlockSpec((1, 4 * _B), lambda g: (0, g)),
            pl.BlockSpec((1, 4 * _B), lambda g: (0, g)),
        ],
        out_specs=[
            pl.BlockSpec((1, _B), lambda g: (0, g)),
            pl.BlockSpec((1, 1), lambda g: (0, 0)),
            pl.BlockSpec((1, 1), lambda g: (0, 0)),
            pl.BlockSpec((1, 1), lambda g: (0, 0)),
        ],
        out_shape=[
            jax.ShapeDtypeStruct((1, _M), jnp.float32),
            jax.ShapeDtypeStruct((1, 1), jnp.float32),
            jax.ShapeDtypeStruct((1, 1), jnp.float32),
            jax.ShapeDtypeStruct((1, 1), jnp.float32),
        ],
    )(conf_t, labels, lpf, ltf, labrep)

    rows = 16
    conf_neg = pl.pallas_call(
        functools.partial(_stage_b_body, rows=rows),
        grid=(_N // rows,),
        in_specs=[pl.BlockSpec((rows, _NB), lambda g: (g, 0))],
        out_specs=pl.BlockSpec((1, 1), lambda g: (0, 0)),
        out_shape=jax.ShapeDtypeStruct((1, 1), jnp.float32),
    )(cl.reshape(_N, _NB))

    nm = num_pos[0, 0]
    total = (loc_loss[0, 0] + nll_pos[0, 0] + conf_neg[0, 0]) / nm
    return jnp.where(nm == 0.0, 0.0, total)
